# baseline (speedup 1.0000x reference)
import functools

import jax
import jax.numpy as jnp
from jax import lax
from jax.experimental import pallas as pl
from jax.experimental.pallas import tpu as pltpu
from jax.experimental.pallas import tpu_sc as plsc

_NC = 2
_NS = 16
_NW = _NC * _NS
_BB = 128
_NB = 3
_L = 16


def _gather_call(batch, hist, hidden):
    mesh = plsc.VectorSubcoreMesh(core_axis_name="c", subcore_axis_name="s")
    per_w = batch // _NW
    nblk_b = per_w // _BB
    steps = hist * nblk_b
    dh = hidden // 8
    blk_elems = hidden * _BB

    @functools.partial(
        pl.kernel,
        mesh=mesh,
        compiler_params=pltpu.CompilerParams(
            use_tc_tiling_on_sc=False, needs_layout_passes=False
        ),
        out_type=jax.ShapeDtypeStruct((hist, dh, batch // _BB, 8, _BB), jnp.float32),
        scratch_types=[
            pltpu.VMEM((per_w * hist,), jnp.int32),
            pltpu.VMEM((_NB, _BB), jnp.int32),
            pltpu.VMEM((_NB, _BB, hidden), jnp.float32),
            pltpu.VMEM((_NB, dh, 8, _BB), jnp.float32),
            pltpu.SemaphoreType.DMA,
            pltpu.SemaphoreType.DMA,
            pltpu.SemaphoreType.DMA,
            pltpu.SemaphoreType.DMA,
            pltpu.SemaphoreType.DMA,
            pltpu.SemaphoreType.DMA,
        ],
    )
    def run(idx_hbm, tab_hbm, out_hbm, idx_v, idxl, rows, trans, *sems):
        gsem = sems[:_NB]
        ssem = sems[_NB:]
        wid = lax.axis_index("s") * _NC + lax.axis_index("c")
        b0w = wid * per_w
        pltpu.sync_copy(idx_hbm.at[pl.ds(b0w * hist, per_w * hist)], idx_v)

        iota = lax.iota(jnp.int32, _L)
        iota_h = iota * hist
        iota_r = iota

        def build_idx(k, p):
            t = k // nblk_b
            bb = (k % nblk_b) * _BB
            for v in range(_BB // _L):
                pos = iota_h + ((bb + v * _L) * hist + t)
                idxl[p, pl.ds(v * _L, _L)] = plsc.load_gather(idx_v, [pos])

        def fire_gather(k, p):
            build_idx(k, p)
            pltpu.async_copy(tab_hbm.at[idxl.at[p]], rows.at[p], gsem[p])

        def drain_gather(p):
            pltpu.make_async_copy(
                tab_hbm.at[idxl.at[0]], rows.at[p], gsem[p]
            ).wait()

        bvecs = [iota_r + v * _L for v in range(_BB // _L)]

        def transpose(p):
            rp = rows.at[p]

            def dhloop(dhi, carry):
                dbase = jnp.full((_L,), 8, jnp.int32) * dhi
                for dlo in range(8):
                    dsplat = dbase + dlo
                    for v in range(_BB // _L):
                        g = plsc.load_gather(rp, [bvecs[v], dsplat])
                        trans[p, dhi, dlo, pl.ds(v * _L, _L)] = g
                return carry

            lax.fori_loop(0, dh, dhloop, 0)

        def fire_store(k, p):
            t = k // nblk_b
            col = wid * nblk_b + k % nblk_b
            pltpu.async_copy(
                trans.at[p], out_hbm.at[t, :, col, :, :], ssem[p]
            )

        def drain_store(p):
            pltpu.make_async_copy(
                trans.at[p], out_hbm.at[0, :, 0, :, :], ssem[p]
            ).wait()

        def step(k, p, fire_next, drain_prev):
            if drain_prev:
                drain_store(p)
            if fire_next:
                fire_gather(k + 2, (p + 2) % _NB)
            drain_gather(p)
            transpose(p)
            fire_store(k, p)

        fire_gather(0, 0)
        fire_gather(1, 1)
        step(0, 0, True, False)
        step(1, 1, True, False)
        step(2, 2, True, False)

        def body(it, carry):
            k = it * _NB
            step(k, 0, True, True)
            step(k + 1, 1, True, True)
            step(k + 2, 2, True, True)
            return carry

        n_body = (steps - 3 - 2) // _NB
        lax.fori_loop(1, n_body + 1, body, 0)
        for k in range(3 + n_body * _NB, steps):
            step(k, k % _NB, k + 2 < steps, True)
        for p in range(_NB):
            drain_store(p)

    return run


def kernel(indices, embeddings):
    batch, hist = indices.shape
    num_rows, hidden = embeddings.shape
    idx = indices.astype(jnp.int32).reshape(-1)
    out = _gather_call(batch, hist, hidden)(idx, embeddings)
    out = out.transpose(2, 4, 0, 1, 3)
    return out.reshape(batch, hist, hidden)


# speedup vs baseline: 1.8232x; 1.8232x over previous
import functools

import jax
import jax.numpy as jnp
from jax import lax
from jax.experimental import pallas as pl
from jax.experimental.pallas import tpu as pltpu
from jax.experimental.pallas import tpu_sc as plsc

_NC = 2
_NS = 16
_NW = _NC * _NS
_BB = 128
_NB = 3
_L = 16


def _gather_call(batch, hist, hidden):
    mesh = plsc.VectorSubcoreMesh(core_axis_name="c", subcore_axis_name="s")
    per_w = batch // _NW
    nblk_b = per_w // _BB
    steps = hist * nblk_b
    dh = hidden // 8
    blk_elems = hidden * _BB

    @functools.partial(
        pl.kernel,
        mesh=mesh,
        compiler_params=pltpu.CompilerParams(
            use_tc_tiling_on_sc=False, needs_layout_passes=False
        ),
        out_type=jax.ShapeDtypeStruct((hist, dh, batch // _BB, 8, _BB), jnp.float32),
        scratch_types=[
            pltpu.VMEM((per_w * hist,), jnp.int32),
            pltpu.VMEM((_NB, _BB), jnp.int32),
            pltpu.VMEM((_NB, _BB, hidden), jnp.float32),
            pltpu.VMEM((_NB, dh, 8, _BB), jnp.float32),
            pltpu.SemaphoreType.DMA,
            pltpu.SemaphoreType.DMA,
            pltpu.SemaphoreType.DMA,
            pltpu.SemaphoreType.DMA,
            pltpu.SemaphoreType.DMA,
            pltpu.SemaphoreType.DMA,
        ],
    )
    def run(idx_hbm, tab_hbm, out_hbm, idx_v, idxl, rows, trans, *sems):
        gsem = sems[:_NB]
        ssem = sems[_NB:]
        wid = lax.axis_index("s") * _NC + lax.axis_index("c")
        b0w = wid * per_w
        pltpu.sync_copy(idx_hbm.at[pl.ds(b0w * hist, per_w * hist)], idx_v)

        iota = lax.iota(jnp.int32, _L)
        iota_h = iota * hist
        iota_r = iota

        def build_idx(k, p):
            t = k // nblk_b
            bb = (k % nblk_b) * _BB
            for v in range(_BB // _L):
                pos = iota_h + ((bb + v * _L) * hist + t)
                idxl[p, pl.ds(v * _L, _L)] = plsc.load_gather(idx_v, [pos])

        def fire_gather(k, p):
            build_idx(k, p)
            pltpu.async_copy(tab_hbm.at[idxl.at[p]], rows.at[p], gsem[p])

        def drain_gather(p):
            pltpu.make_async_copy(
                tab_hbm.at[idxl.at[0]], rows.at[p], gsem[p]
            ).wait()

        bvecs = [iota_r + v * _L for v in range(_BB // _L)]

        def transpose(p):
            rp = rows.at[p]
            tp = trans.at[p]

            def jloop(j, carry):
                dvec = (iota_r + j) & (hidden - 1)
                dhi_v = lax.shift_right_logical(dvec, 3)
                dlo_v = dvec & 7
                for v in range(_BB // _L):
                    g = plsc.load_gather(rp, [bvecs[v], dvec])
                    plsc.store_scatter(tp, [dhi_v, dlo_v, bvecs[v]], g)
                return carry

            lax.fori_loop(0, hidden, jloop, 0)

        def fire_store(k, p):
            t = k // nblk_b
            col = wid * nblk_b + k % nblk_b
            pltpu.async_copy(
                trans.at[p], out_hbm.at[t, :, col, :, :], ssem[p]
            )

        def drain_store(p):
            pltpu.make_async_copy(
                trans.at[p], out_hbm.at[0, :, 0, :, :], ssem[p]
            ).wait()

        def step(k, p, fire_next, drain_prev):
            if drain_prev:
                drain_store(p)
            if fire_next:
                fire_gather(k + 2, (p + 2) % _NB)
            drain_gather(p)
            transpose(p)
            fire_store(k, p)

        fire_gather(0, 0)
        fire_gather(1, 1)
        step(0, 0, True, False)
        step(1, 1, True, False)
        step(2, 2, True, False)

        def body(it, carry):
            k = it * _NB
            step(k, 0, True, True)
            step(k + 1, 1, True, True)
            step(k + 2, 2, True, True)
            return carry

        n_body = (steps - 3 - 2) // _NB
        lax.fori_loop(1, n_body + 1, body, 0)
        for k in range(3 + n_body * _NB, steps):
            step(k, k % _NB, k + 2 < steps, True)
        for p in range(_NB):
            drain_store(p)

    return run


def kernel(indices, embeddings):
    batch, hist = indices.shape
    num_rows, hidden = embeddings.shape
    idx = indices.astype(jnp.int32).reshape(-1)
    out = _gather_call(batch, hist, hidden)(idx, embeddings)
    out = out.transpose(2, 4, 0, 1, 3)
    return out.reshape(batch, hist, hidden)


# speedup vs baseline: 2.4698x; 1.3547x over previous
import functools

import jax
import jax.numpy as jnp
from jax import lax
from jax.experimental import pallas as pl
from jax.experimental.pallas import tpu as pltpu
from jax.experimental.pallas import tpu_sc as plsc

_NC = 2
_NS = 16
_NW = _NC * _NS
_BB = 128
_NB = 3
_L = 16


def _gather_call(batch, hist, hidden):
    mesh = plsc.VectorSubcoreMesh(core_axis_name="c", subcore_axis_name="s")
    per_w = batch // _NW
    nblk_b = per_w // _BB
    steps = hist * nblk_b
    dh = hidden // 8
    blk_elems = hidden * _BB

    @functools.partial(
        pl.kernel,
        mesh=mesh,
        compiler_params=pltpu.CompilerParams(
            use_tc_tiling_on_sc=False, needs_layout_passes=False
        ),
        out_type=jax.ShapeDtypeStruct((hist, dh, batch // _BB, 8, _BB), jnp.float32),
        scratch_types=[
            pltpu.VMEM((per_w * hist,), jnp.int32),
            pltpu.VMEM((_NB, _BB), jnp.int32),
            pltpu.VMEM((_NB, _BB, hidden), jnp.float32),
            pltpu.VMEM((_NB, dh, 8, _BB), jnp.float32),
            pltpu.SemaphoreType.DMA,
            pltpu.SemaphoreType.DMA,
            pltpu.SemaphoreType.DMA,
            pltpu.SemaphoreType.DMA,
            pltpu.SemaphoreType.DMA,
            pltpu.SemaphoreType.DMA,
        ],
    )
    def run(idx_hbm, tab_hbm, out_hbm, idx_v, idxl, rows, trans, *sems):
        gsem = sems[:_NB]
        ssem = sems[_NB:]
        wid = lax.axis_index("s") * _NC + lax.axis_index("c")
        b0w = wid * per_w
        pltpu.sync_copy(idx_hbm.at[pl.ds(b0w * hist, per_w * hist)], idx_v)

        iota = lax.iota(jnp.int32, _L)
        iota_h = iota * hist
        iota_r = iota

        def build_idx(k, p):
            t = k // nblk_b
            bb = (k % nblk_b) * _BB
            for v in range(_BB // _L):
                pos = iota_h + ((bb + v * _L) * hist + t)
                idxl[p, pl.ds(v * _L, _L)] = plsc.load_gather(idx_v, [pos])

        def fire_gather(k, p):
            build_idx(k, p)
            pltpu.async_copy(tab_hbm.at[idxl.at[p]], rows.at[p], gsem[p])

        def drain_gather(p):
            pltpu.make_async_copy(
                tab_hbm.at[idxl.at[0]], rows.at[p], gsem[p]
            ).wait()

        bvecs = [iota_r + v * _L for v in range(_BB // _L)]

        def transpose(p):
            rp = rows.at[p]
            tp = trans.at[p]

            def jloop(j, carry):
                for u in range(2):
                    dvec = (iota_r + (j * 2 + u)) & (hidden - 1)
                    dhi_v = lax.shift_right_logical(dvec, 3)
                    dlo_v = dvec & 7
                    gs = [
                        plsc.load_gather(rp, [bvecs[v], dvec])
                        for v in range(_BB // _L)
                    ]
                    for v in range(_BB // _L):
                        plsc.store_scatter(tp, [dhi_v, dlo_v, bvecs[v]], gs[v])
                return carry

            lax.fori_loop(0, hidden // 2, jloop, 0)

        def fire_store(k, p):
            t = k // nblk_b
            col = wid * nblk_b + k % nblk_b
            pltpu.async_copy(
                trans.at[p], out_hbm.at[t, :, col, :, :], ssem[p]
            )

        def drain_store(p):
            pltpu.make_async_copy(
                trans.at[p], out_hbm.at[0, :, 0, :, :], ssem[p]
            ).wait()

        def step(k, p, fire_next, drain_prev):
            if drain_prev:
                drain_store(p)
            if fire_next:
                fire_gather(k + 2, (p + 2) % _NB)
            drain_gather(p)
            transpose(p)
            fire_store(k, p)

        fire_gather(0, 0)
        fire_gather(1, 1)
        step(0, 0, True, False)
        step(1, 1, True, False)
        step(2, 2, True, False)

        def body(it, carry):
            k = it * _NB
            step(k, 0, True, True)
            step(k + 1, 1, True, True)
            step(k + 2, 2, True, True)
            return carry

        n_body = (steps - 3 - 2) // _NB
        lax.fori_loop(1, n_body + 1, body, 0)
        for k in range(3 + n_body * _NB, steps):
            step(k, k % _NB, k + 2 < steps, True)
        for p in range(_NB):
            drain_store(p)

    return run


def kernel(indices, embeddings):
    batch, hist = indices.shape
    num_rows, hidden = embeddings.shape
    idx = indices.astype(jnp.int32).reshape(-1)
    out = _gather_call(batch, hist, hidden)(idx, embeddings)
    out = out.transpose(2, 4, 0, 1, 3)
    return out.reshape(batch, hist, hidden)


# speedup vs baseline: 2.4846x; 1.0060x over previous
import functools

import jax
import jax.numpy as jnp
from jax import lax
from jax.experimental import pallas as pl
from jax.experimental.pallas import tpu as pltpu
from jax.experimental.pallas import tpu_sc as plsc

_NC = 2
_NS = 16
_NW = _NC * _NS
_BB = 128
_NB = 3
_L = 16


def _gather_call(batch, hist, hidden):
    mesh = plsc.VectorSubcoreMesh(core_axis_name="c", subcore_axis_name="s")
    per_w = batch // _NW
    nblk_b = per_w // _BB
    steps = hist * nblk_b
    dh = hidden // 8

    @functools.partial(
        pl.kernel,
        mesh=mesh,
        compiler_params=pltpu.CompilerParams(
            use_tc_tiling_on_sc=False, needs_layout_passes=False
        ),
        out_type=jax.ShapeDtypeStruct((hist, dh, batch // _BB, 8, _BB), jnp.float32),
        scratch_types=[
            pltpu.VMEM((per_w * hist,), jnp.int32),
            pltpu.VMEM((_NB, _BB), jnp.int32),
            pltpu.VMEM((_NB, _BB, hidden), jnp.float32),
            pltpu.VMEM((_NB, dh, 8, _BB), jnp.float32),
            pltpu.SemaphoreType.DMA,
            pltpu.SemaphoreType.DMA,
            pltpu.SemaphoreType.DMA,
            pltpu.SemaphoreType.DMA,
            pltpu.SemaphoreType.DMA,
            pltpu.SemaphoreType.DMA,
        ],
    )
    def run(idx_hbm, tab_hbm, out_hbm, idx_v, idxl, rows, trans, *sems):
        gsem = sems[:_NB]
        ssem = sems[_NB:]
        wid = lax.axis_index("s") * _NC + lax.axis_index("c")
        b0w = wid * per_w
        pltpu.sync_copy(idx_hbm.at[pl.ds(b0w * hist, per_w * hist)], idx_v)

        iota = lax.iota(jnp.int32, _L)
        iota_h = iota * hist
        iota_r = iota

        def build_idx(k, p):
            t = k // nblk_b
            bb = (k % nblk_b) * _BB
            for v in range(_BB // _L):
                pos = iota_h + ((bb + v * _L) * hist + t)
                idxl[p, pl.ds(v * _L, _L)] = plsc.load_gather(idx_v, [pos])

        def fire_gather(k, p):
            build_idx(k, p)
            pltpu.async_copy(tab_hbm.at[idxl.at[p]], rows.at[p], gsem[p])

        def drain_gather(p):
            pltpu.make_async_copy(
                tab_hbm.at[idxl.at[0]], rows.at[p], gsem[p]
            ).wait()

        bvecs = [iota_r + v * _L for v in range(_BB // _L)]

        def transpose(p):
            rp = rows.at[p]
            tp = trans.at[p]

            def jloop(j, carry):
                for u in range(2):
                    dvec = (iota_r + (j * 2 + u)) & (hidden - 1)
                    dhi_v = lax.shift_right_logical(dvec, 3)
                    dlo_v = dvec & 7
                    gs = [
                        plsc.load_gather(rp, [bvecs[v], dvec])
                        for v in range(_BB // _L)
                    ]
                    for v in range(_BB // _L):
                        plsc.store_scatter(tp, [dhi_v, dlo_v, bvecs[v]], gs[v])
                return carry

            lax.fori_loop(0, hidden // 2, jloop, 0)

        def fire_store(k, p):
            t = k // nblk_b
            col = wid * nblk_b + k % nblk_b
            pltpu.async_copy(
                trans.at[p], out_hbm.at[t, :, col, :, :], ssem[p]
            )

        def drain_store(p):
            pltpu.make_async_copy(
                trans.at[p], out_hbm.at[0, :, 0, :, :], ssem[p]
            ).wait()

        def step(k, p, fire_next, drain_prev):
            if drain_prev:
                drain_store(p)
            if fire_next:
                fire_gather(k + 2, (p + 2) % _NB)
            drain_gather(p)
            transpose(p)
            fire_store(k, p)

        fire_gather(0, 0)
        fire_gather(1, 1)
        step(0, 0, True, False)
        step(1, 1, True, False)
        step(2, 2, True, False)

        def body(it, carry):
            k = it * _NB
            step(k, 0, True, True)
            step(k + 1, 1, True, True)
            step(k + 2, 2, True, True)
            return carry

        n_body = (steps - 3 - 2) // _NB
        lax.fori_loop(1, n_body + 1, body, 0)
        for k in range(3 + n_body * _NB, steps):
            step(k, k % _NB, k + 2 < steps, True)
        for p in range(_NB):
            drain_store(p)

    return run


def kernel(indices, embeddings):
    batch, hist = indices.shape
    num_rows, hidden = embeddings.shape
    idx = indices.astype(jnp.int32).reshape(-1)
    out = _gather_call(batch, hist, hidden)(idx, embeddings)
    out = out.transpose(2, 4, 0, 1, 3)
    return out.reshape(batch, hist, hidden)
